# baseline (speedup 1.0000x reference)
import functools

import jax
import jax.numpy as jnp
from jax import lax
from jax.experimental import pallas as pl
from jax.experimental.pallas import tpu as pltpu
from jax.experimental.pallas import tpu_sc as plsc

E = 64
K = 8


def _tc_body(nblk, bt, x_ref, wt_ref,
             logits_ref, probs_ref, topi_ref, ranks_ref, tpe_ref, cnt_ref,
             lb_ref, zl_ref, ltri_ref, psum_ref):
    step = pl.program_id(0)
    ntok = nblk * bt

    sub = 128

    @pl.when(step == 0)
    def _init():
        tpe_ref[...] = jnp.zeros_like(tpe_ref)
        psum_ref[...] = jnp.zeros_like(psum_ref)
        zl_ref[...] = jnp.zeros_like(zl_ref)
        ra = lax.broadcasted_iota(jnp.int32, (sub, sub), 0)
        cb = lax.broadcasted_iota(jnp.int32, (sub, sub), 1)
        ltri_ref[...] = (cb < ra).astype(jnp.bfloat16)

    logits = jnp.dot(x_ref[...].astype(jnp.bfloat16), wt_ref[...],
                     preferred_element_type=jnp.float32)
    logits_ref[...] = logits

    ltri = ltri_ref[...]
    run = tpe_ref[...]
    psum_acc = jnp.zeros((1, E), jnp.float32)
    zl_acc = jnp.zeros((1, 1), jnp.float32)

    lanes = lax.broadcasted_iota(jnp.int32, (sub, E), 1)

    for cix in range(bt // sub):
        lg = logits[cix * sub:(cix + 1) * sub, :]
        work = jnp.clip(lg, -2.0, 2.0)
        onehot = jnp.zeros((sub, E), jnp.float32)
        sels, vals, idxs = [], [], []
        for _ in range(K):
            m = jnp.max(work, axis=1, keepdims=True)
            ismax = work == m
            idx = jnp.min(jnp.where(ismax, lanes, E), axis=1, keepdims=True)
            self32 = (lanes == idx).astype(jnp.float32)
            vals.append(m)
            idxs.append(idx)
            sels.append(self32)
            onehot = onehot + self32
            work = work - self32 * (work + 3.0)
        topv = jnp.concatenate(vals, axis=1)
        topi_ref[cix * sub:(cix + 1) * sub, :] = jnp.concatenate(idxs, axis=1)

        ex = jnp.exp(topv - topv[:, :1])
        probs_ref[cix * sub:(cix + 1) * sub, :] = (
            ex / jnp.sum(ex, axis=1, keepdims=True))

        csum = jnp.dot(ltri, onehot.astype(jnp.bfloat16),
                       preferred_element_type=jnp.float32)
        total = csum + run
        rcols = [jnp.sum(sels[k] * total, axis=1, keepdims=True)
                 for k in range(K)]
        ranks_ref[cix * sub:(cix + 1) * sub, :] = (
            jnp.concatenate(rcols, axis=1).astype(jnp.int32))
        run = run + jnp.sum(onehot, axis=0, keepdims=True)

        mfull = jnp.max(lg, axis=1, keepdims=True)
        p = jnp.exp(lg - mfull)
        s = jnp.sum(p, axis=1, keepdims=True)
        psum_acc = psum_acc + jnp.sum(p / s, axis=0, keepdims=True)
        lse = jnp.log(s) + mfull
        zl_acc = zl_acc + jnp.sum(lse * lse).reshape(1, 1)

    tpe_ref[...] = run
    psum_ref[...] = psum_ref[...] + psum_acc
    zl_ref[...] = zl_ref[...] + zl_acc

    @pl.when(step == nblk - 1)
    def _fini():
        cnt = tpe_ref[...]
        cnt_ref[...] = cnt.astype(jnp.int32)
        lb = jnp.sum(cnt * psum_ref[...]) * (float(E) / (ntok * K) / ntok)
        lb_ref[...] = lb.reshape(1, 1)
        zl_ref[...] = zl_ref[...] * (1.0 / ntok)


def _tc_router(x, wt16, bt=512):
    t, h = x.shape
    nblk = t // bt
    f32, i32 = jnp.float32, jnp.int32
    return pl.pallas_call(
        functools.partial(_tc_body, nblk, bt),
        grid=(nblk,),
        in_specs=[
            pl.BlockSpec((bt, h), lambda i: (i, 0)),
            pl.BlockSpec((h, E), lambda i: (0, 0)),
        ],
        out_specs=[
            pl.BlockSpec((bt, E), lambda i: (i, 0)),
            pl.BlockSpec((bt, K), lambda i: (i, 0)),
            pl.BlockSpec((bt, K), lambda i: (i, 0)),
            pl.BlockSpec((bt, K), lambda i: (i, 0)),
            pl.BlockSpec((1, E), lambda i: (0, 0)),
            pl.BlockSpec((1, E), lambda i: (0, 0)),
            pl.BlockSpec((1, 1), lambda i: (0, 0)),
            pl.BlockSpec((1, 1), lambda i: (0, 0)),
        ],
        out_shape=[
            jax.ShapeDtypeStruct((t, E), f32),
            jax.ShapeDtypeStruct((t, K), f32),
            jax.ShapeDtypeStruct((t, K), i32),
            jax.ShapeDtypeStruct((t, K), i32),
            jax.ShapeDtypeStruct((1, E), f32),
            jax.ShapeDtypeStruct((1, E), i32),
            jax.ShapeDtypeStruct((1, 1), f32),
            jax.ShapeDtypeStruct((1, 1), f32),
        ],
        scratch_shapes=[
            pltpu.VMEM((128, 128), jnp.bfloat16),
            pltpu.VMEM((1, E), jnp.float32),
        ],
    )(x, wt16)


def _sc_body(n, e_hbm, r_hbm, cnt_hbm, ar_hbm, out_hbm, grp_hbm,
             e_v, r_v, val_v, idx_v, cnt_v, grp_v, start_v, sh, sem):
    c = lax.axis_index("c")
    s = lax.axis_index("s")
    ch = n // 16
    tbase = s * ch

    pltpu.sync_copy(e_hbm.at[pl.ds(tbase, ch)], e_v)
    pltpu.sync_copy(r_hbm.at[pl.ds(tbase, ch)], r_v)
    pltpu.sync_copy(ar_hbm.at[pl.ds(tbase, ch)], val_v)
    pltpu.sync_copy(cnt_hbm, cnt_v)

    carry = jnp.int32(0)
    for q in range(E // 16):
        v = cnt_v[pl.ds(q * 16, 16)]
        g = plsc.cumsum(v) + carry
        grp_v[pl.ds(q * 16, 16)] = g
        start_v[pl.ds(q * 16, 16)] = g - v
        carry = carry + jnp.sum(v)

    @pl.when(jnp.logical_and(c == 0, s == 0))
    def _():
        pltpu.sync_copy(grp_v, grp_hbm)

    @plsc.parallel_loop(0, ch // 16, unroll=16)
    def _fill(g):
        ev = e_v[pl.ds(g * 16, 16)]
        rv = r_v[pl.ds(g * 16, 16)]
        idx_v[pl.ds(g * 16, 16)] = plsc.load_gather(start_v, [ev]) + rv

    pltpu.async_copy(val_v, sh.at[idx_v], sem).wait()
    plsc.subcore_barrier()

    half = n // 2
    per = half // 16
    off = c * half + s * per
    pltpu.sync_copy(sh.at[pl.ds(off, per)], out_hbm.at[pl.ds(off, per)])


def _sc_sort(e_flat, r_flat, cnt, ar):
    n = e_flat.shape[0]
    ch = n // 16
    i32 = jnp.int32
    mesh = plsc.VectorSubcoreMesh(core_axis_name="c", subcore_axis_name="s")
    return pl.kernel(
        functools.partial(_sc_body, n),
        out_type=[
            jax.ShapeDtypeStruct((n,), i32),
            jax.ShapeDtypeStruct((E,), i32),
        ],
        mesh=mesh,
        compiler_params=pltpu.CompilerParams(needs_layout_passes=False),
        scratch_types=[
            pltpu.VMEM((ch,), i32),
            pltpu.VMEM((ch,), i32),
            pltpu.VMEM((ch,), i32),
            pltpu.VMEM((ch,), i32),
            pltpu.VMEM((E,), i32),
            pltpu.VMEM((E,), i32),
            pltpu.VMEM((E,), i32),
            pltpu.VMEM_SHARED((n,), i32),
            pltpu.SemaphoreType.DMA,
        ],
    )(e_flat, r_flat, cnt, ar)


def kernel(x, W):
    wt16 = W.T.astype(jnp.bfloat16)
    (logits, probs, topi, ranks, tpe, cnt, lb, zl) = _tc_router(x, wt16)
    ar = lax.iota(jnp.int32, topi.size)
    indices, grp = _sc_sort(topi.reshape(-1), ranks.reshape(-1),
                            cnt.reshape(E), ar)
    return (logits, probs, topi, grp, indices, tpe.reshape(E),
            lb.reshape(()), zl.reshape(()))


# speedup vs baseline: 1.0220x; 1.0220x over previous
import functools

import jax
import jax.numpy as jnp
from jax import lax
from jax.experimental import pallas as pl
from jax.experimental.pallas import tpu as pltpu
from jax.experimental.pallas import tpu_sc as plsc

E = 64
K = 8


def _tc_body(nblk, bt, x_ref, wt_ref,
             logits_ref, probs_ref, topi_ref, ranks_ref, tpe_ref, cnt_ref,
             lb_ref, zl_ref, ltri_ref, psum_ref):
    step = pl.program_id(0)
    ntok = nblk * bt

    sub = 128

    @pl.when(step == 0)
    def _init():
        tpe_ref[...] = jnp.zeros_like(tpe_ref)
        psum_ref[...] = jnp.zeros_like(psum_ref)
        zl_ref[...] = jnp.zeros_like(zl_ref)
        ra = lax.broadcasted_iota(jnp.int32, (sub, sub), 0)
        cb = lax.broadcasted_iota(jnp.int32, (sub, sub), 1)
        ltri_ref[...] = (cb < ra).astype(jnp.bfloat16)

    logits = jnp.dot(x_ref[...].astype(jnp.bfloat16), wt_ref[...],
                     preferred_element_type=jnp.float32)
    logits_ref[...] = logits

    ltri = ltri_ref[...]
    run = tpe_ref[...]
    psum_acc = jnp.zeros((1, E), jnp.float32)
    zl_acc = jnp.zeros((1, 1), jnp.float32)

    lanes = lax.broadcasted_iota(jnp.int32, (sub, E), 1)

    for cix in range(bt // sub):
        lg = logits[cix * sub:(cix + 1) * sub, :]
        work = jnp.clip(lg, -2.0, 2.0)
        onehot = jnp.zeros((sub, E), jnp.float32)
        sels, vals, idxs = [], [], []
        for _ in range(K):
            m = jnp.max(work, axis=1, keepdims=True)
            ismax = work == m
            idx = jnp.min(jnp.where(ismax, lanes, E), axis=1, keepdims=True)
            self32 = (lanes == idx).astype(jnp.float32)
            vals.append(m)
            idxs.append(idx)
            sels.append(self32)
            onehot = onehot + self32
            work = work - self32 * (work + 3.0)
        topv = jnp.concatenate(vals, axis=1)
        topi_ref[cix * sub:(cix + 1) * sub, :] = jnp.concatenate(idxs, axis=1)

        ex = jnp.exp(topv - topv[:, :1])
        probs_ref[cix * sub:(cix + 1) * sub, :] = (
            ex / jnp.sum(ex, axis=1, keepdims=True))

        csum = jnp.dot(ltri, onehot.astype(jnp.bfloat16),
                       preferred_element_type=jnp.float32)
        total = csum + run
        rcols = [jnp.sum(sels[k] * total, axis=1, keepdims=True)
                 for k in range(K)]
        ranks_ref[cix * sub:(cix + 1) * sub, :] = (
            jnp.concatenate(rcols, axis=1).astype(jnp.int32))
        run = run + jnp.sum(onehot, axis=0, keepdims=True)

        mfull = jnp.max(lg, axis=1, keepdims=True)
        p = jnp.exp(lg - mfull)
        s = jnp.sum(p, axis=1, keepdims=True)
        psum_acc = psum_acc + jnp.sum(p / s, axis=0, keepdims=True)
        lse = jnp.log(s) + mfull
        zl_acc = zl_acc + jnp.sum(lse * lse).reshape(1, 1)

    tpe_ref[...] = run
    psum_ref[...] = psum_ref[...] + psum_acc
    zl_ref[...] = zl_ref[...] + zl_acc

    @pl.when(step == nblk - 1)
    def _fini():
        cnt = tpe_ref[...]
        cnt_ref[...] = cnt.astype(jnp.int32)
        lb = jnp.sum(cnt * psum_ref[...]) * (float(E) / (ntok * K) / ntok)
        lb_ref[...] = lb.reshape(1, 1)
        zl_ref[...] = zl_ref[...] * (1.0 / ntok)


def _tc_router(x, wt16, bt=1024):
    t, h = x.shape
    nblk = t // bt
    f32, i32 = jnp.float32, jnp.int32
    return pl.pallas_call(
        functools.partial(_tc_body, nblk, bt),
        grid=(nblk,),
        in_specs=[
            pl.BlockSpec((bt, h), lambda i: (i, 0)),
            pl.BlockSpec((h, E), lambda i: (0, 0)),
        ],
        out_specs=[
            pl.BlockSpec((bt, E), lambda i: (i, 0)),
            pl.BlockSpec((bt, K), lambda i: (i, 0)),
            pl.BlockSpec((bt, K), lambda i: (i, 0)),
            pl.BlockSpec((bt, K), lambda i: (i, 0)),
            pl.BlockSpec((1, E), lambda i: (0, 0)),
            pl.BlockSpec((1, E), lambda i: (0, 0)),
            pl.BlockSpec((1, 1), lambda i: (0, 0)),
            pl.BlockSpec((1, 1), lambda i: (0, 0)),
        ],
        out_shape=[
            jax.ShapeDtypeStruct((t, E), f32),
            jax.ShapeDtypeStruct((t, K), f32),
            jax.ShapeDtypeStruct((t, K), i32),
            jax.ShapeDtypeStruct((t, K), i32),
            jax.ShapeDtypeStruct((1, E), f32),
            jax.ShapeDtypeStruct((1, E), i32),
            jax.ShapeDtypeStruct((1, 1), f32),
            jax.ShapeDtypeStruct((1, 1), f32),
        ],
        scratch_shapes=[
            pltpu.VMEM((128, 128), jnp.bfloat16),
            pltpu.VMEM((1, E), jnp.float32),
        ],
    )(x, wt16)


def _sc_body(n, e_hbm, r_hbm, cnt_hbm, ar_hbm, out_hbm, grp_hbm,
             e_v, r_v, val_v, idx_v, cnt_v, grp_v, start_v, sh, sem):
    c = lax.axis_index("c")
    s = lax.axis_index("s")
    ch = n // 16
    tbase = s * ch

    pltpu.sync_copy(e_hbm.at[pl.ds(tbase, ch)], e_v)
    pltpu.sync_copy(r_hbm.at[pl.ds(tbase, ch)], r_v)
    pltpu.sync_copy(ar_hbm.at[pl.ds(tbase, ch)], val_v)
    pltpu.sync_copy(cnt_hbm, cnt_v)

    carry = jnp.int32(0)
    for q in range(E // 16):
        v = cnt_v[pl.ds(q * 16, 16)]
        g = plsc.cumsum(v) + carry
        grp_v[pl.ds(q * 16, 16)] = g
        start_v[pl.ds(q * 16, 16)] = g - v
        carry = carry + jnp.sum(v)

    @pl.when(jnp.logical_and(c == 0, s == 0))
    def _():
        pltpu.sync_copy(grp_v, grp_hbm)

    @plsc.parallel_loop(0, ch // 16, unroll=16)
    def _fill(g):
        ev = e_v[pl.ds(g * 16, 16)]
        rv = r_v[pl.ds(g * 16, 16)]
        idx_v[pl.ds(g * 16, 16)] = plsc.load_gather(start_v, [ev]) + rv

    pltpu.async_copy(val_v, sh.at[idx_v], sem).wait()
    plsc.subcore_barrier()

    half = n // 2
    per = half // 16
    off = c * half + s * per
    pltpu.sync_copy(sh.at[pl.ds(off, per)], out_hbm.at[pl.ds(off, per)])


def _sc_sort(e_flat, r_flat, cnt, ar):
    n = e_flat.shape[0]
    ch = n // 16
    i32 = jnp.int32
    mesh = plsc.VectorSubcoreMesh(core_axis_name="c", subcore_axis_name="s")
    return pl.kernel(
        functools.partial(_sc_body, n),
        out_type=[
            jax.ShapeDtypeStruct((n,), i32),
            jax.ShapeDtypeStruct((E,), i32),
        ],
        mesh=mesh,
        compiler_params=pltpu.CompilerParams(needs_layout_passes=False),
        scratch_types=[
            pltpu.VMEM((ch,), i32),
            pltpu.VMEM((ch,), i32),
            pltpu.VMEM((ch,), i32),
            pltpu.VMEM((ch,), i32),
            pltpu.VMEM((E,), i32),
            pltpu.VMEM((E,), i32),
            pltpu.VMEM((E,), i32),
            pltpu.VMEM_SHARED((n,), i32),
            pltpu.SemaphoreType.DMA,
        ],
    )(e_flat, r_flat, cnt, ar)


def kernel(x, W):
    wt16 = W.T.astype(jnp.bfloat16)
    (logits, probs, topi, ranks, tpe, cnt, lb, zl) = _tc_router(x, wt16)
    ar = lax.iota(jnp.int32, topi.size)
    indices, grp = _sc_sort(topi.reshape(-1), ranks.reshape(-1),
                            cnt.reshape(E), ar)
    return (logits, probs, topi, grp, indices, tpe.reshape(E),
            lb.reshape(()), zl.reshape(()))
